# baseline (speedup 1.0000x reference)
import functools

import jax
import jax.numpy as jnp
from jax import lax
from jax.experimental import pallas as pl
from jax.experimental.pallas import tpu as pltpu
from jax.experimental.pallas import tpu_sc as plsc

_LANES = 16
_NW = 32
_CHUNK = 128


def _ln_embed_sc(word_f, posi_f, age_f, gender_f, word_table, posi_table,
                 age_table, gender_table, ln_gamma, ln_beta):
    n_tok = word_f.shape[0]
    hid = word_table.shape[1]
    regs = hid // _LANES
    per_w = n_tok // _NW
    n_chunks = per_w // _CHUNK

    mesh = plsc.VectorSubcoreMesh(core_axis_name="c", subcore_axis_name="s")

    @functools.partial(
        pl.kernel,
        mesh=mesh,
        out_type=jax.ShapeDtypeStruct((n_tok, hid), jnp.float32),
        scratch_types=[
            pltpu.VMEM((_CHUNK,), jnp.int32),
            pltpu.VMEM((_CHUNK,), jnp.int32),
            pltpu.VMEM((_CHUNK,), jnp.int32),
            pltpu.VMEM((_CHUNK,), jnp.int32),
            pltpu.VMEM((_CHUNK, hid), jnp.float32),
            pltpu.VMEM((_CHUNK, hid), jnp.float32),
            pltpu.VMEM((_CHUNK, hid), jnp.float32),
            pltpu.VMEM((_CHUNK, hid), jnp.float32),
            pltpu.VMEM((hid,), jnp.float32),
            pltpu.VMEM((hid,), jnp.float32),
            pltpu.SemaphoreType.DMA,
        ],
        compiler_params=pltpu.CompilerParams(use_tc_tiling_on_sc=False),
    )
    def k(w_hbm, p_hbm, a_hbm, g_hbm, wt_hbm, pt_hbm, at_hbm, gt_hbm,
          gam_hbm, bet_hbm, out_hbm,
          wi_v, pi_v, ai_v, gi_v, wb_v, pb_v, ab_v, gb_v, gam_v, bet_v, sem):
        cid = lax.axis_index("c")
        sid = lax.axis_index("s")
        wid = sid * 2 + cid
        base_w = wid * per_w

        pltpu.sync_copy(gam_hbm, gam_v)
        pltpu.sync_copy(bet_hbm, bet_v)
        gam_r = [gam_v[pl.ds(i * _LANES, _LANES)] for i in range(regs)]
        bet_r = [bet_v[pl.ds(i * _LANES, _LANES)] for i in range(regs)]

        lane = lax.iota(jnp.int32, _LANES)
        bfly = [lax.bitwise_xor(lane, jnp.int32(1 << p)) for p in range(4)]
        _dnums = lax.GatherDimensionNumbers(
            offset_dims=(), collapsed_slice_dims=(0,), start_index_map=(0,))

        def lane_perm(v, perm):
            return lax.gather(
                v, perm[:, None], _dnums, slice_sizes=(1,),
                mode=lax.GatherScatterMode.PROMISE_IN_BOUNDS)

        def lane_allsum(v):
            for perm in bfly:
                v = v + lane_perm(v, perm)
            return v

        def chunk_body(j, carry):
            base = base_w + j * _CHUNK
            pltpu.sync_copy(w_hbm.at[pl.ds(base, _CHUNK)], wi_v)
            pltpu.sync_copy(p_hbm.at[pl.ds(base, _CHUNK)], pi_v)
            pltpu.sync_copy(a_hbm.at[pl.ds(base, _CHUNK)], ai_v)
            pltpu.sync_copy(g_hbm.at[pl.ds(base, _CHUNK)], gi_v)

            cw = pltpu.async_copy(wt_hbm.at[wi_v], wb_v, sem)
            cp = pltpu.async_copy(pt_hbm.at[pi_v], pb_v, sem)
            ca = pltpu.async_copy(at_hbm.at[ai_v], ab_v, sem)
            cg = pltpu.async_copy(gt_hbm.at[gi_v], gb_v, sem)
            cw.wait()
            cp.wait()
            ca.wait()
            cg.wait()

            def tok(t, c2):
                x = [wb_v[t, pl.ds(i * _LANES, _LANES)]
                     + pb_v[t, pl.ds(i * _LANES, _LANES)]
                     + ab_v[t, pl.ds(i * _LANES, _LANES)]
                     + gb_v[t, pl.ds(i * _LANES, _LANES)]
                     for i in range(regs)]
                tot = x[0]
                for xi in x[1:]:
                    tot = tot + xi
                mean = lane_allsum(tot) * (1.0 / hid)
                cen = [xi - mean for xi in x]
                sq = cen[0] * cen[0]
                for ci in cen[1:]:
                    sq = sq + ci * ci
                vb = lane_allsum(sq) * (1.0 / hid) + 1e-12
                bits = lax.bitcast_convert_type(vb, jnp.int32)
                y = lax.bitcast_convert_type(
                    jnp.int32(0x5F3759DF) - lax.shift_right_arithmetic(bits, 1),
                    jnp.float32)
                for _ in range(3):
                    y = y * (1.5 - 0.5 * vb * y * y)
                for i in range(regs):
                    wb_v[t, pl.ds(i * _LANES, _LANES)] = (
                        cen[i] * y * gam_r[i] + bet_r[i])
                return c2

            lax.fori_loop(0, _CHUNK, tok, 0, unroll=2)
            pltpu.sync_copy(wb_v, out_hbm.at[pl.ds(base, _CHUNK)])
            return carry

        lax.fori_loop(0, n_chunks, chunk_body, 0)

    return k(word_f, posi_f, age_f, gender_f, word_table, posi_table,
             age_table, gender_table, ln_gamma, ln_beta)


def kernel(word_ids, posi_ids, age_ids, gender_ids, word_table, posi_table,
           age_table, gender_table, ln_gamma, ln_beta):
    b, l = word_ids.shape
    hid = word_table.shape[1]
    n_tok = b * l
    out = _ln_embed_sc(
        word_ids.reshape(n_tok).astype(jnp.int32),
        posi_ids.reshape(n_tok).astype(jnp.int32),
        age_ids.reshape(n_tok).astype(jnp.int32),
        gender_ids.reshape(n_tok).astype(jnp.int32),
        word_table, posi_table, age_table, gender_table, ln_gamma, ln_beta)
    return out.reshape(b, l, hid)


# speedup vs baseline: 2.8003x; 2.8003x over previous
import functools

import jax
import jax.numpy as jnp
from jax import lax
from jax.experimental import pallas as pl
from jax.experimental.pallas import tpu as pltpu
from jax.experimental.pallas import tpu_sc as plsc

_LANES = 16
_NW = 32
_CHUNK = 128
_GRP = _CHUNK // _LANES


def _ln_embed_sc(ids3, word_table, posi_table, ag_table, ln_gamma, ln_beta):
    n_tok = ids3.shape[1]
    hid = word_table.shape[1]
    n_posi = posi_table.shape[0]
    n_ag = ag_table.shape[0]
    per_w = n_tok // _NW
    n_chunks = per_w // _CHUNK

    mesh = plsc.VectorSubcoreMesh(core_axis_name="c", subcore_axis_name="s")

    @functools.partial(
        pl.kernel,
        mesh=mesh,
        out_type=jax.ShapeDtypeStruct((n_tok, hid), jnp.float32),
        scratch_types=[
            pltpu.VMEM((3, _CHUNK), jnp.int32),
            pltpu.VMEM((_CHUNK, hid), jnp.float32),
            pltpu.VMEM((_CHUNK, hid), jnp.float32),
            pltpu.VMEM((_GRP, hid, _LANES), jnp.float32),
            pltpu.VMEM((n_posi, hid), jnp.float32),
            pltpu.VMEM((n_ag, hid), jnp.float32),
            pltpu.VMEM((hid,), jnp.float32),
            pltpu.VMEM((hid,), jnp.float32),
            pltpu.SemaphoreType.DMA,
        ],
        compiler_params=pltpu.CompilerParams(use_tc_tiling_on_sc=False,
                                             needs_layout_passes=False),
    )
    def k(ids_hbm, wt_hbm, pt_hbm, agt_hbm, gam_hbm, bet_hbm, out_hbm,
          iv, wb, ob, xs, pt_v, agt_v, gam_v, bet_v, sem):
        cid = lax.axis_index("c")
        sid = lax.axis_index("s")
        wid = sid * 2 + cid
        base_w = wid * per_w

        pltpu.sync_copy(pt_hbm, pt_v)
        pltpu.sync_copy(agt_hbm, agt_v)
        pltpu.sync_copy(gam_hbm, gam_v)
        pltpu.sync_copy(bet_hbm, bet_v)

        lane = lax.iota(jnp.int32, _LANES)

        def chunk_body(c, carry):
            base = base_w + c * _CHUNK
            pltpu.sync_copy(ids_hbm.at[:, pl.ds(base, _CHUNK)], iv)
            pltpu.async_copy(wt_hbm.at[iv.at[0]], wb, sem).wait()

            scale = []
            shift = []
            tokl = []
            for g in range(_GRP):
                tl = lane + g * _LANES
                tokl.append(tl)
                pidx = iv[1, pl.ds(g * _LANES, _LANES)]
                agidx = iv[2, pl.ds(g * _LANES, _LANES)]
                zeros = jnp.zeros((_LANES,), jnp.float32)

                def jstep(jj, acc):
                    s0, s1 = acc
                    js = jnp.full((_LANES,), jj, jnp.int32)
                    x = (plsc.load_gather(wb, [tl, js])
                         + plsc.load_gather(pt_v, [pidx, js])
                         + plsc.load_gather(agt_v, [agidx, js]))
                    xs[g, jj, :] = x
                    return (s0 + x, s1 + x * x)

                s0, s1 = lax.fori_loop(0, hid, jstep, (zeros, zeros),
                                       unroll=4)
                mean = s0 * (1.0 / hid)
                var = s1 * (1.0 / hid) - mean * mean + 1e-12
                bits = lax.bitcast_convert_type(var, jnp.int32)
                y = lax.bitcast_convert_type(
                    jnp.int32(0x5F3759DF) - lax.shift_right_arithmetic(bits, 1),
                    jnp.float32)
                for _ in range(3):
                    y = y * (1.5 - 0.5 * var * y * y)
                scale.append(y)
                shift.append(-mean * y)

            def jnorm(jj, carry2):
                js = jnp.full((_LANES,), jj, jnp.int32)
                gam = plsc.load_gather(gam_v, [js])
                bet = plsc.load_gather(bet_v, [js])
                for g in range(_GRP):
                    x = xs[g, jj, :]
                    o = (x * scale[g] + shift[g]) * gam + bet
                    plsc.store_scatter(ob, [tokl[g], js], o)
                return carry2

            lax.fori_loop(0, hid, jnorm, 0, unroll=2)
            pltpu.sync_copy(ob, out_hbm.at[pl.ds(base, _CHUNK)])
            return carry

        lax.fori_loop(0, n_chunks, chunk_body, 0)

    return k(ids3, word_table, posi_table, ag_table, ln_gamma, ln_beta)


def kernel(word_ids, posi_ids, age_ids, gender_ids, word_table, posi_table,
           age_table, gender_table, ln_gamma, ln_beta):
    b, l = word_ids.shape
    hid = word_table.shape[1]
    n_tok = b * l
    n_gen = gender_table.shape[0]
    ids3 = jnp.stack([
        word_ids.reshape(n_tok).astype(jnp.int32),
        posi_ids.reshape(n_tok).astype(jnp.int32),
        (age_ids.reshape(n_tok) * n_gen + gender_ids.reshape(n_tok)
         ).astype(jnp.int32),
    ])
    ag_table = (age_table[:, None, :] + gender_table[None, :, :]
                ).reshape(-1, hid)
    out = _ln_embed_sc(ids3, word_table, posi_table, ag_table,
                       ln_gamma, ln_beta)
    return out.reshape(b, l, hid)


# speedup vs baseline: 2.9579x; 1.0563x over previous
import functools

import jax
import jax.numpy as jnp
from jax import lax
from jax.experimental import pallas as pl
from jax.experimental.pallas import tpu as pltpu
from jax.experimental.pallas import tpu_sc as plsc

_LANES = 16
_NW = 32
_CHUNK = 128
_GRP = _CHUNK // _LANES


def _ln_embed_sc(ids3, word_table, posi_table, ag_table, ln_gamma, ln_beta):
    n_tok = ids3.shape[1]
    hid = word_table.shape[1]
    n_posi = posi_table.shape[0]
    n_ag = ag_table.shape[0]
    per_w = n_tok // _NW
    n_chunks = per_w // _CHUNK

    mesh = plsc.VectorSubcoreMesh(core_axis_name="c", subcore_axis_name="s")

    @functools.partial(
        pl.kernel,
        mesh=mesh,
        out_type=jax.ShapeDtypeStruct((n_tok, hid), jnp.float32),
        scratch_types=[
            [pltpu.VMEM((3, _CHUNK), jnp.int32) for _ in range(4)],
            [pltpu.VMEM((_CHUNK, hid), jnp.float32) for _ in range(4)],
            [pltpu.VMEM((_CHUNK, hid), jnp.float32) for _ in range(2)],
            pltpu.VMEM((_GRP, hid, _LANES), jnp.float32),
            pltpu.VMEM((n_posi, hid), jnp.float32),
            pltpu.VMEM((n_ag, hid), jnp.float32),
            pltpu.VMEM((hid,), jnp.float32),
            pltpu.VMEM((hid,), jnp.float32),
            [pltpu.SemaphoreType.DMA for _ in range(4)],
            [pltpu.SemaphoreType.DMA for _ in range(4)],
            [pltpu.SemaphoreType.DMA for _ in range(2)],
        ],
        compiler_params=pltpu.CompilerParams(use_tc_tiling_on_sc=False,
                                             needs_layout_passes=False),
    )
    def k(ids_hbm, wt_hbm, pt_hbm, agt_hbm, gam_hbm, bet_hbm, out_hbm,
          iv, wb, ob, xs, pt_v, agt_v, gam_v, bet_v, isem, gsem, osem):
        cid = lax.axis_index("c")
        sid = lax.axis_index("s")
        wid = sid * 2 + cid
        base_w = wid * per_w

        pltpu.sync_copy(pt_hbm, pt_v)
        pltpu.sync_copy(agt_hbm, agt_v)
        pltpu.sync_copy(gam_hbm, gam_v)
        pltpu.sync_copy(bet_hbm, bet_v)

        lane = lax.iota(jnp.int32, _LANES)

        def idx_copy(c, u):
            return pltpu.make_async_copy(
                ids_hbm.at[:, pl.ds(base_w + c * _CHUNK, _CHUNK)],
                iv[u], isem[u])

        def gather(c, u):
            return pltpu.make_async_copy(wt_hbm.at[iv[u].at[0]],
                                         wb[u], gsem[u])

        def scatter(c, u):
            return pltpu.make_async_copy(
                ob[u % 2], out_hbm.at[pl.ds(base_w + c * _CHUNK, _CHUNK)],
                osem[u % 2])

        def compute(c, u):
            scale = []
            shift = []
            tokl = []
            for g in range(_GRP):
                tl = lane + g * _LANES
                tokl.append(tl)
                pidx = iv[u][1, pl.ds(g * _LANES, _LANES)]
                agidx = iv[u][2, pl.ds(g * _LANES, _LANES)]
                zeros = jnp.zeros((_LANES,), jnp.float32)

                def jstep(jj, acc, g=g, tl=tl, pidx=pidx, agidx=agidx):
                    s0, s1 = acc
                    js = jnp.full((_LANES,), jj, jnp.int32)
                    x = (plsc.load_gather(wb[u], [tl, js])
                         + plsc.load_gather(pt_v, [pidx, js])
                         + plsc.load_gather(agt_v, [agidx, js]))
                    xs[g, jj, :] = x
                    return (s0 + x, s1 + x * x)

                s0, s1 = lax.fori_loop(0, hid, jstep, (zeros, zeros),
                                       unroll=4)
                mean = s0 * (1.0 / hid)
                var = s1 * (1.0 / hid) - mean * mean + 1e-12
                bits = lax.bitcast_convert_type(var, jnp.int32)
                y = lax.bitcast_convert_type(
                    jnp.int32(0x5F3759DF) - lax.shift_right_arithmetic(bits, 1),
                    jnp.float32)
                for _ in range(3):
                    y = y * (1.5 - 0.5 * var * y * y)
                scale.append(y)
                shift.append(-mean * y)

            def jnorm(jj, carry2):
                js = jnp.full((_LANES,), jj, jnp.int32)
                gam = plsc.load_gather(gam_v, [js])
                bet = plsc.load_gather(bet_v, [js])
                for g in range(_GRP):
                    x = xs[g, jj, :]
                    o = (x * scale[g] + shift[g]) * gam + bet
                    plsc.store_scatter(ob[u % 2], [tokl[g], js], o)
                return carry2

            lax.fori_loop(0, hid, jnorm, 0, unroll=2)

        for u in range(4):
            idx_copy(u, u).start()
        for u in range(2):
            idx_copy(u, u).wait()
            gather(u, u).start()

        def quad_body(t, carry):
            for u in range(4):
                c = t * 4 + u
                gather(c, u).wait()

                @pl.when(c + 2 < n_chunks)
                def _(u=u, c=c):
                    idx_copy(c + 2, (u + 2) % 4).wait()
                    gather(c + 2, (u + 2) % 4).start()

                @pl.when(c >= 2)
                def _(u=u, c=c):
                    scatter(c - 2, u).wait()

                compute(c, u)
                scatter(c, u).start()

                @pl.when(c + 4 < n_chunks)
                def _(u=u, c=c):
                    idx_copy(c + 4, u).start()
            return carry

        lax.fori_loop(0, n_chunks // 4, quad_body, 0)
        scatter(n_chunks - 2, 2).wait()
        scatter(n_chunks - 1, 3).wait()

    return k(ids3, word_table, posi_table, ag_table, ln_gamma, ln_beta)


def kernel(word_ids, posi_ids, age_ids, gender_ids, word_table, posi_table,
           age_table, gender_table, ln_gamma, ln_beta):
    b, l = word_ids.shape
    hid = word_table.shape[1]
    n_tok = b * l
    n_gen = gender_table.shape[0]
    ids3 = jnp.stack([
        word_ids.reshape(n_tok).astype(jnp.int32),
        posi_ids.reshape(n_tok).astype(jnp.int32),
        (age_ids.reshape(n_tok) * n_gen + gender_ids.reshape(n_tok)
         ).astype(jnp.int32),
    ])
    ag_table = (age_table[:, None, :] + gender_table[None, :, :]
                ).reshape(-1, hid)
    out = _ln_embed_sc(ids3, word_table, posi_table, ag_table,
                       ln_gamma, ln_beta)
    return out.reshape(b, l, hid)


# speedup vs baseline: 6.2749x; 2.1214x over previous
import functools

import jax
import jax.numpy as jnp
from jax import lax
from jax.experimental import pallas as pl
from jax.experimental.pallas import tpu as pltpu
from jax.experimental.pallas import tpu_sc as plsc

_LANES = 16
_NW = 32
_CHUNK = 128
_GRP = _CHUNK // _LANES


def _ln_embed_sc(ids3, word_table, posi_table, ag_table, ln_gamma, ln_beta):
    n_tok = ids3.shape[1]
    hid = word_table.shape[1]
    n_posi = posi_table.shape[0]
    n_ag = ag_table.shape[0]
    per_w = n_tok // _NW
    n_chunks = per_w // _CHUNK

    mesh = plsc.VectorSubcoreMesh(core_axis_name="c", subcore_axis_name="s")

    @functools.partial(
        pl.kernel,
        mesh=mesh,
        out_type=jax.ShapeDtypeStruct((n_tok, hid), jnp.float32),
        scratch_types=[
            [pltpu.VMEM((3, _CHUNK), jnp.int32) for _ in range(4)],
            [pltpu.VMEM((_CHUNK, hid), jnp.float32) for _ in range(4)],
            [pltpu.VMEM((_CHUNK, hid), jnp.float32) for _ in range(2)],
            pltpu.VMEM((_GRP * hid * _LANES,), jnp.float32),
            pltpu.VMEM((n_posi, hid), jnp.float32),
            pltpu.VMEM((n_ag, hid), jnp.float32),
            pltpu.VMEM((hid,), jnp.float32),
            pltpu.VMEM((hid,), jnp.float32),
            [pltpu.SemaphoreType.DMA for _ in range(4)],
            [pltpu.SemaphoreType.DMA for _ in range(4)],
            [pltpu.SemaphoreType.DMA for _ in range(2)],
        ],
        compiler_params=pltpu.CompilerParams(use_tc_tiling_on_sc=False,
                                             needs_layout_passes=False),
    )
    def k(ids_hbm, wt_hbm, pt_hbm, agt_hbm, gam_hbm, bet_hbm, out_hbm,
          iv, wb, ob, xs, pt_v, agt_v, gam_v, bet_v, isem, gsem, osem):
        cid = lax.axis_index("c")
        sid = lax.axis_index("s")
        wid = sid * 2 + cid
        base_w = wid * per_w

        pltpu.sync_copy(pt_hbm, pt_v)
        pltpu.sync_copy(agt_hbm, agt_v)
        pltpu.sync_copy(gam_hbm, gam_v)
        pltpu.sync_copy(bet_hbm, bet_v)

        lane = lax.iota(jnp.int32, _LANES)

        def idx_copy(c, u):
            return pltpu.make_async_copy(
                ids_hbm.at[:, pl.ds(base_w + c * _CHUNK, _CHUNK)],
                iv[u], isem[u])

        def gather(c, u):
            return pltpu.make_async_copy(wt_hbm.at[iv[u].at[0]],
                                         wb[u], gsem[u])

        def scatter(c, u):
            return pltpu.make_async_copy(
                ob[u % 2], out_hbm.at[pl.ds(base_w + c * _CHUNK, _CHUNK)],
                osem[u % 2])

        def compute(c, u):
            scale = []
            shift = []
            tokl = []
            gxl = []
            for g in range(_GRP):
                tl = lane + g * _LANES
                tokl.append(tl)
                gxl.append(lane + g * (hid * _LANES))
                pidx = iv[u][1, pl.ds(g * _LANES, _LANES)]
                agidx = iv[u][2, pl.ds(g * _LANES, _LANES)]
                zeros = jnp.zeros((_LANES,), jnp.float32)

                def jstep(jj, acc, g=g, tl=tl, pidx=pidx, agidx=agidx):
                    s0, s1 = acc
                    e = lax.bitwise_and(lane + jj, hid - 1)
                    x = (plsc.load_gather(wb[u], [tl, e])
                         + plsc.load_gather(pt_v, [pidx, e])
                         + plsc.load_gather(agt_v, [agidx, e]))
                    plsc.store_scatter(
                        xs, [gxl[g] + lax.shift_left(e, 4)], x)
                    return (s0 + x, s1 + x * x)

                s0, s1 = lax.fori_loop(0, hid, jstep, (zeros, zeros),
                                       unroll=4)
                mean = s0 * (1.0 / hid)
                var = s1 * (1.0 / hid) - mean * mean + 1e-12
                bits = lax.bitcast_convert_type(var, jnp.int32)
                y = lax.bitcast_convert_type(
                    jnp.int32(0x5F3759DF) - lax.shift_right_arithmetic(bits, 1),
                    jnp.float32)
                for _ in range(3):
                    y = y * (1.5 - 0.5 * var * y * y)
                scale.append(y)
                shift.append(-mean * y)

            def jnorm(jj, carry2):
                e = lax.bitwise_and(lane + jj, hid - 1)
                gam = plsc.load_gather(gam_v, [e])
                bet = plsc.load_gather(bet_v, [e])
                e16 = lax.shift_left(e, 4)
                for g in range(_GRP):
                    x = plsc.load_gather(xs, [gxl[g] + e16])
                    o = (x * scale[g] + shift[g]) * gam + bet
                    plsc.store_scatter(ob[u % 2], [tokl[g], e], o)
                return carry2

            lax.fori_loop(0, hid, jnorm, 0, unroll=2)

        for u in range(4):
            idx_copy(u, u).start()
        for u in range(2):
            idx_copy(u, u).wait()
            gather(u, u).start()

        def quad_body(t, carry):
            for u in range(4):
                c = t * 4 + u
                gather(c, u).wait()

                @pl.when(c + 2 < n_chunks)
                def _(u=u, c=c):
                    idx_copy(c + 2, (u + 2) % 4).wait()
                    gather(c + 2, (u + 2) % 4).start()

                @pl.when(c >= 2)
                def _(u=u, c=c):
                    scatter(c - 2, u).wait()

                compute(c, u)
                scatter(c, u).start()

                @pl.when(c + 4 < n_chunks)
                def _(u=u, c=c):
                    idx_copy(c + 4, u).start()
            return carry

        lax.fori_loop(0, n_chunks // 4, quad_body, 0)
        scatter(n_chunks - 2, 2).wait()
        scatter(n_chunks - 1, 3).wait()

    return k(ids3, word_table, posi_table, ag_table, ln_gamma, ln_beta)


def kernel(word_ids, posi_ids, age_ids, gender_ids, word_table, posi_table,
           age_table, gender_table, ln_gamma, ln_beta):
    b, l = word_ids.shape
    hid = word_table.shape[1]
    n_tok = b * l
    n_gen = gender_table.shape[0]
    ids3 = jnp.stack([
        word_ids.reshape(n_tok).astype(jnp.int32),
        posi_ids.reshape(n_tok).astype(jnp.int32),
        (age_ids.reshape(n_tok) * n_gen + gender_ids.reshape(n_tok)
         ).astype(jnp.int32),
    ])
    ag_table = (age_table[:, None, :] + gender_table[None, :, :]
                ).reshape(-1, hid)
    out = _ln_embed_sc(ids3, word_table, posi_table, ag_table,
                       ln_gamma, ln_beta)
    return out.reshape(b, l, hid)
